# baseline (speedup 1.0000x reference)
import functools

import jax
import jax.numpy as jnp
from jax import lax
from jax.experimental import pallas as pl
from jax.experimental.pallas import tpu as pltpu
from jax.experimental.pallas import tpu_sc as plsc

S = 200
B = 4096
D = 64
L = 16

NC = 2
NS = 16
NW = NC * NS
BW = B // NW


def _body(src_hbm, table_hbm, pe_hbm, out_hbm, idx_v, pe_v, rows_v, gsem):
    wid = lax.axis_index("s") * NC + lax.axis_index("c")
    bcol = wid * BW

    pltpu.sync_copy(src_hbm.at[:, pl.ds(bcol, BW)], idx_v)
    pltpu.sync_copy(pe_hbm, pe_v)

    def step(s, carry):
        pltpu.async_copy(table_hbm.at[idx_v.at[s]], rows_v, gsem).wait()

        pe0 = pe_v[s, pl.ds(0, L)]
        pe1 = pe_v[s, pl.ds(L, L)]
        pe2 = pe_v[s, pl.ds(2 * L, L)]
        pe3 = pe_v[s, pl.ds(3 * L, L)]

        def add_row(i, c):
            rows_v[i, pl.ds(0, L)] = rows_v[i, pl.ds(0, L)] + pe0
            rows_v[i, pl.ds(L, L)] = rows_v[i, pl.ds(L, L)] + pe1
            rows_v[i, pl.ds(2 * L, L)] = rows_v[i, pl.ds(2 * L, L)] + pe2
            rows_v[i, pl.ds(3 * L, L)] = rows_v[i, pl.ds(3 * L, L)] + pe3
            return c

        lax.fori_loop(0, BW, add_row, 0)

        pltpu.sync_copy(rows_v, out_hbm.at[s, pl.ds(bcol, BW)])
        return carry

    lax.fori_loop(0, S, step, 0)


@jax.jit
def _pe_embed(src, table, pe2d):
    mesh = plsc.VectorSubcoreMesh(core_axis_name="c", subcore_axis_name="s")
    k = pl.kernel(
        _body,
        out_type=jax.ShapeDtypeStruct((S, B, D), jnp.float32),
        mesh=mesh,
        scratch_types=[
            pltpu.VMEM((S, BW), jnp.int32),
            pltpu.VMEM((S, D), jnp.float32),
            pltpu.VMEM((BW, D), jnp.float32),
            pltpu.SemaphoreType.DMA,
        ],
        compiler_params=pltpu.CompilerParams(use_tc_tiling_on_sc=False),
    )
    return k(src, table, pe2d)


def kernel(src, table, pe):
    src = src.astype(jnp.int32)
    pe2d = pe.reshape(S, D)
    return _pe_embed(src, table, pe2d)
